# Initial kernel scaffold; baseline (speedup 1.0000x reference)
import jax
import jax.numpy as jnp
from jax.experimental import pallas as pl


def kernel(x):
    raise NotImplementedError("write your pallas kernel here")


# speedup vs baseline: 648.0709x; 648.0709x over previous
import jax
import jax.numpy as jnp
from jax.experimental import pallas as pl
from jax.experimental.pallas import tpu as pltpu

_GRID = 8
_BLK_C = 65536 // _GRID


def _nan_free_body(x_ref, out_ref):
    i = pl.program_id(0)
    blk = x_ref[...]
    ok = jnp.where(jnp.any(blk != blk), 0, 1).astype(jnp.int32)

    @pl.when(i == 0)
    def _init():
        out_ref[0, 0] = ok

    @pl.when(i > 0)
    def _acc():
        out_ref[0, 0] = jnp.minimum(out_ref[0, 0], ok)


@jax.jit
def kernel(x):
    ok = pl.pallas_call(
        _nan_free_body,
        grid=(_GRID,),
        in_specs=[pl.BlockSpec((8, 32, _BLK_C), lambda i: (0, 0, i))],
        out_specs=pl.BlockSpec(
            block_shape=(1, 1),
            index_map=lambda i: (0, 0),
            memory_space=pltpu.SMEM,
        ),
        out_shape=jax.ShapeDtypeStruct((1, 1), jnp.int32),
    )(x)
    return ok[0, 0].astype(jnp.bool_)
